# baseline (speedup 1.0000x reference)
import functools

import jax
import jax.numpy as jnp
from jax import lax
from jax.experimental import pallas as pl
from jax.experimental.pallas import tpu as pltpu
from jax.experimental.pallas import tpu_sc as plsc
from jax._src.pallas import mpmd as plsc_mpmd
from jax._src.pallas import core as pallas_core

_NUM_CORES = 2
_NUM_SUBCORES = 16

_VEC_ROWS = 3584
_VEC_CHUNK = 16
_VEC_NBUF = 2

_SCS_CHUNK = 256
_SCS_NBUF = 4


def _ring_copy(table_hbm, out_hbm, buf, sems_in, sems_out, base, rows, chunk,
               nbuf):
    nchunks = rows // chunk
    assert nchunks * chunk == rows

    def start_in(j):
        b = j % nbuf
        return pltpu.async_copy(
            table_hbm.at[pl.ds(base + j * chunk, chunk)],
            buf.at[b],
            sems_in.at[b],
        )

    def start_out(j):
        b = j % nbuf
        return pltpu.async_copy(
            buf.at[b],
            out_hbm.at[pl.ds(base + j * chunk, chunk)],
            sems_out.at[b],
        )

    in_flight = [start_in(j) for j in range(min(nbuf, nchunks))]
    out_flight = [None] * nchunks
    for j in range(nchunks):
        in_flight[j].wait()
        out_flight[j] = start_out(j)
        if j + nbuf < nchunks:
            out_flight[j].wait()
            in_flight.append(start_in(j + nbuf))
    for j in range(max(0, nchunks - nbuf), nchunks):
        out_flight[j].wait()


def _scs_sem_array(mesh, n):
    return pallas_core.CoreMemorySpace(pltpu.MemorySpace.SEMAPHORE, mesh)(
        (n,), pltpu.SemaphoreType.DMA.dtype
    )


def _positional_rows(pos_table, seq_len):
    d_model = pos_table.shape[1]
    dtype = pos_table.dtype

    vec_rows = _VEC_ROWS
    scs_rows = seq_len - vec_rows
    num_workers = _NUM_CORES * _NUM_SUBCORES
    vec_rows_per_w = vec_rows // num_workers
    scs_rows_per_w = scs_rows // _NUM_CORES
    assert vec_rows_per_w * num_workers == vec_rows
    assert scs_rows_per_w * _NUM_CORES == scs_rows

    scalar_mesh = plsc.ScalarSubcoreMesh(axis_name="c", num_cores=_NUM_CORES)
    vector_mesh = plsc.VectorSubcoreMesh(
        core_axis_name="c", subcore_axis_name="s"
    )

    def scs_fn(table_hbm, out_hbm, buf, sems_in, sems_out):
        if scs_rows:
            wid = lax.axis_index("c")
            base = vec_rows + wid * scs_rows_per_w
            _ring_copy(table_hbm, out_hbm, buf, sems_in, sems_out, base,
                       scs_rows_per_w, _SCS_CHUNK, _SCS_NBUF)

    def tec_fn(table_hbm, out_hbm, scs_buf, scs_sems_in, scs_sems_out):
        del scs_buf, scs_sems_in, scs_sems_out
        wid = lax.axis_index("s") * _NUM_CORES + lax.axis_index("c")
        base = wid * vec_rows_per_w

        def body(buf, sems_in, sems_out):
            _ring_copy(table_hbm, out_hbm, buf, sems_in, sems_out, base,
                       vec_rows_per_w, _VEC_CHUNK, _VEC_NBUF)

        pl.run_scoped(
            body,
            pltpu.VMEM((_VEC_NBUF, _VEC_CHUNK, d_model), dtype),
            pltpu.SemaphoreType.DMA((_VEC_NBUF,)),
            pltpu.SemaphoreType.DMA((_VEC_NBUF,)),
        )

    copy_rows = plsc_mpmd.mpmd_map(
        [(scalar_mesh, scs_fn), (vector_mesh, tec_fn)],
        out_types=jax.ShapeDtypeStruct((seq_len, d_model), dtype),
        scratch_types=[
            pltpu.VMEM_SHARED((_SCS_NBUF, _SCS_CHUNK, d_model), dtype),
            _scs_sem_array(scalar_mesh, _SCS_NBUF),
            _scs_sem_array(scalar_mesh, _SCS_NBUF),
        ],
    )
    return copy_rows(pos_table)


def kernel(x, pos_table):
    seq_len = x.shape[1]
    return _positional_rows(pos_table, seq_len)[None]


# speedup vs baseline: 1.0427x; 1.0427x over previous
import functools

import jax
import jax.numpy as jnp
from jax import lax
from jax.experimental import pallas as pl
from jax.experimental.pallas import tpu as pltpu
from jax.experimental.pallas import tpu_sc as plsc

_NUM_CORES = 2
_NUM_SUBCORES = 16
_CHUNK_ROWS = 64
_NBUF = 2


def _positional_rows(pos_table, seq_len):
    d_model = pos_table.shape[1]
    num_workers = _NUM_CORES * _NUM_SUBCORES
    rows_per_w = seq_len // num_workers
    assert rows_per_w * num_workers == seq_len
    assert rows_per_w % _CHUNK_ROWS == 0
    nchunks = rows_per_w // _CHUNK_ROWS

    mesh = plsc.VectorSubcoreMesh(core_axis_name="c", subcore_axis_name="s")

    @functools.partial(
        pl.kernel,
        out_type=jax.ShapeDtypeStruct((seq_len, d_model), pos_table.dtype),
        mesh=mesh,
        scratch_types=[
            pltpu.VMEM((_NBUF, _CHUNK_ROWS, d_model), pos_table.dtype),
            pltpu.SemaphoreType.DMA((_NBUF,)),
            pltpu.SemaphoreType.DMA((_NBUF,)),
        ],
    )
    def copy_rows(table_hbm, out_hbm, buf, sems_in, sems_out):
        wid = lax.axis_index("s") * _NUM_CORES + lax.axis_index("c")
        base = wid * rows_per_w

        def start_in(j):
            b = j % _NBUF
            return pltpu.async_copy(
                table_hbm.at[pl.ds(base + j * _CHUNK_ROWS, _CHUNK_ROWS)],
                buf.at[b],
                sems_in.at[b],
            )

        def start_out(j):
            b = j % _NBUF
            return pltpu.async_copy(
                buf.at[b],
                out_hbm.at[pl.ds(base + j * _CHUNK_ROWS, _CHUNK_ROWS)],
                sems_out.at[b],
            )

        in_flight = [start_in(j) for j in range(min(_NBUF, nchunks))]
        out_flight = [None] * nchunks
        for j in range(nchunks):
            in_flight[j].wait()
            out_flight[j] = start_out(j)
            if j + _NBUF < nchunks:
                out_flight[j].wait()
                in_flight.append(start_in(j + _NBUF))
        for j in range(max(0, nchunks - _NBUF), nchunks):
            out_flight[j].wait()

    return copy_rows(pos_table)


def kernel(x, pos_table):
    seq_len = x.shape[1]
    return _positional_rows(pos_table, seq_len)[None]
